# baseline (speedup 1.0000x reference)
import functools

import jax
import jax.numpy as jnp
import numpy as np
from jax import lax
from jax.experimental import pallas as pl
from jax.experimental.pallas import tpu as pltpu
from jax.experimental.pallas import tpu_sc as plsc

_SIGMA = np.empty((128,), dtype=np.int32)
for _j in range(4):
    for _k in range(16):
        _SIGMA[32 * _j + 2 * _k] = 32 * _j + _k
        _SIGMA[32 * _j + 2 * _k + 1] = 32 * _j + 16 + _k


def _pack_bf16(a):
    r = a.shape[0]
    return lax.bitcast_convert_type(a.reshape(r, 64, 2), jnp.int32)


def _node_body(x_ref, xsa_ref, xsb_ref, wda_ref, wsa_ref, wdb_ref, wsb_ref,
               hda_ref, hsa_ref, hdb_ref, hsb_ref):
    x = x_ref[...]
    hda_ref[...] = jnp.dot(x, wda_ref[...], preferred_element_type=jnp.float32)
    hdb_ref[...] = jnp.dot(x, wdb_ref[...], preferred_element_type=jnp.float32)
    hsa_ref[...] = jnp.dot(xsa_ref[...], wsa_ref[...],
                           preferred_element_type=jnp.float32)
    hsb_ref[...] = jnp.dot(xsb_ref[...], wsb_ref[...],
                           preferred_element_type=jnp.float32)


def _node_transform(x, xs_a, xs_b, Wda, Wsa, Wdb, Wsb):
    n, d = x.shape
    bn = 1000
    grid = (n // bn,)
    row_spec = pl.BlockSpec((bn, d), lambda i: (i, 0))
    w_spec = pl.BlockSpec((d, d), lambda i: (0, 0))
    out = jax.ShapeDtypeStruct((n, d), jnp.float32)
    return pl.pallas_call(
        _node_body,
        grid=grid,
        in_specs=[row_spec, row_spec, row_spec, w_spec, w_spec, w_spec, w_spec],
        out_specs=[row_spec, row_spec, row_spec, row_spec],
        out_shape=[out, out, out, out],
    )(x, xs_a, xs_b, Wda, Wsa, Wdb, Wsb)


def _edge_body(aa_ref, ab_ref, wea_ref, web_ref, ba_ref, bb_ref,
               ea_ref, eb_ref):
    ea_ref[...] = (jnp.dot(aa_ref[...], wea_ref[...],
                           preferred_element_type=jnp.float32)
                   + ba_ref[...]).astype(jnp.bfloat16)
    eb_ref[...] = (jnp.dot(ab_ref[...], web_ref[...],
                           preferred_element_type=jnp.float32)
                   + bb_ref[...]).astype(jnp.bfloat16)


def _edge_transform(attr_a, attr_b, Wea, Web, ba, bb):
    e, de = attr_a.shape
    d = Wea.shape[1]
    be = 8000
    grid = (e // be,)
    a_spec = pl.BlockSpec((be, de), lambda i: (i, 0))
    w_spec = pl.BlockSpec((de, d), lambda i: (0, 0))
    b_spec = pl.BlockSpec((1, d), lambda i: (0, 0))
    o_spec = pl.BlockSpec((be, d), lambda i: (i, 0))
    out = jax.ShapeDtypeStruct((e, d), jnp.bfloat16)
    return pl.pallas_call(
        _edge_body,
        grid=grid,
        in_specs=[a_spec, a_spec, w_spec, w_spec, b_spec, b_spec],
        out_specs=[o_spec, o_spec],
        out_shape=[out, out],
    )(attr_a, attr_b, Wea, Web, ba.reshape(1, d), bb.reshape(1, d))


def _final_body(acc_ref, wua_ref, wub_ref, bu_ref, out_ref):
    msg_a = acc_ref[0, 0] + acc_ref[0, 1]
    msg_b = acc_ref[1, 0] + acc_ref[1, 1]
    out_ref[...] = (
        jnp.dot(msg_a, wua_ref[...], preferred_element_type=jnp.float32)
        + jnp.dot(msg_b, wub_ref[...], preferred_element_type=jnp.float32)
        + bu_ref[...])


def _final_matmul(acc, Wua, Wub, bu):
    n, d = acc.shape[2], acc.shape[3]
    bn = 1000
    grid = (n // bn,)
    return pl.pallas_call(
        _final_body,
        grid=grid,
        in_specs=[
            pl.BlockSpec((2, 2, bn, d), lambda i: (0, 0, i, 0)),
            pl.BlockSpec((d, d), lambda i: (0, 0)),
            pl.BlockSpec((d, d), lambda i: (0, 0)),
            pl.BlockSpec((1, d), lambda i: (0, 0)),
        ],
        out_specs=pl.BlockSpec((bn, d), lambda i: (i, 0)),
        out_shape=jax.ShapeDtypeStruct((n, d), jnp.float32),
    )(acc, Wua, Wub, bu.reshape(1, d))


_CH = 40


def _sc_message_passing(hd_a, hs_a, e_a, dst_a, src_a,
                        hd_b, hs_b, e_b, dst_b, src_b):
    n, d = hd_a.shape
    dp = d // 2
    e = dst_a.shape[0]
    info = plsc.get_sparse_core_info()
    nc, ns, nl = info.num_cores, info.num_subcores, info.num_lanes
    nw = nc * ns
    ept = e // nw
    chunks = ept // _CH
    assert ept * nw == e and chunks * _CH == ept
    zrows = _CH
    units = n // zrows
    assert units * zrows == n
    ng = dp // nl
    nvec = d // nl

    assert chunks >= 4
    nst = 3
    triples = chunks // nst
    rem = chunks % nst

    mesh = plsc.VectorSubcoreMesh(core_axis_name="c", subcore_axis_name="s")

    hbuf_t = pltpu.VMEM((_CH, d), jnp.float32)
    ebuf_t = pltpu.VMEM((_CH, dp), jnp.int32)
    idx_t = pltpu.VMEM((_CH,), jnp.int32)

    @functools.partial(
        pl.kernel,
        mesh=mesh,
        out_type=jax.ShapeDtypeStruct((2, 2, n, d), jnp.float32),
        scratch_types=(
            [idx_t] * (2 * nst)
            + [hbuf_t, hbuf_t, ebuf_t] * nst
            + [pltpu.VMEM_SHARED((n, d), jnp.float32)]
            + [pltpu.SemaphoreType.DMA] * (3 * nst)
        ),
    )
    def sc_kernel(hda, hsa, ea, dsta, srca, hdb, hsb, eb, dstb, srcb,
                  out, *scr):
        idxs = scr[:2 * nst]
        bufs = scr[2 * nst:5 * nst]
        acc_sh = scr[5 * nst]
        sems = scr[5 * nst + 1:]
        stage = tuple(
            (idxs[2 * r], idxs[2 * r + 1],
             bufs[3 * r], bufs[3 * r + 1], bufs[3 * r + 2],
             sems[3 * r], sems[3 * r + 1], sems[3 * r + 2])
            for r in range(nst))
        c = lax.axis_index("c")
        s = lax.axis_index("s")
        wid = s * nc + c

        def _process(t, hd, hs, ee, dst, src):
            def _launch_idx(g, st):
                idx_d, idx_s = st[0], st[1]
                sem_i = st[6]
                base = wid * ept + g * _CH
                pltpu.async_copy(dst.at[pl.ds(base, _CH)], idx_d, sem_i)
                pltpu.async_copy(src.at[pl.ds(base, _CH)], idx_s, sem_i)

            def _wait_idx(st):
                idx_d, idx_s = st[0], st[1]
                sem_i = st[6]
                pltpu.make_async_copy(dst.at[pl.ds(0, _CH)], idx_d,
                                      sem_i).wait()
                pltpu.make_async_copy(src.at[pl.ds(0, _CH)], idx_s,
                                      sem_i).wait()

            def _launch_gather(g, st):
                idx_d, idx_s, hd_buf, hs_buf, e_buf = st[:5]
                sem_g = st[5]
                base = wid * ept + g * _CH
                pltpu.async_copy(hd.at[idx_d], hd_buf, sem_g)
                pltpu.async_copy(hs.at[idx_s], hs_buf, sem_g)
                pltpu.async_copy(ee.at[pl.ds(base, _CH)], e_buf, sem_g)

            def _wait_gather(st):
                idx_d, idx_s, hd_buf, hs_buf, e_buf = st[:5]
                sem_g = st[5]
                pltpu.make_async_copy(hd.at[idx_d], hd_buf, sem_g).wait()
                pltpu.make_async_copy(hs.at[idx_s], hs_buf, sem_g).wait()
                pltpu.make_async_copy(ee.at[pl.ds(0, _CH)], e_buf,
                                      sem_g).wait()

            def _compute_scatter(st):
                idx_d, _, hd_buf, hs_buf, e_buf = st[:5]
                sem_s = st[7]

                hmask = jnp.full((nl,), -65536, jnp.int32)

                def _f32s(w):
                    lo = lax.bitcast_convert_type(
                        jnp.left_shift(w, 16), jnp.float32)
                    hi = lax.bitcast_convert_type(
                        jnp.bitwise_and(w, hmask), jnp.float32)
                    return lo, hi

                def _row(i, _):
                    for j in range(ng):
                        e_lo, e_hi = _f32s(e_buf[i, pl.ds(j * nl, nl)])
                        slo = pl.ds(2 * j * nl, nl)
                        shi = pl.ds((2 * j + 1) * nl, nl)
                        hd_buf[i, slo] = jnp.maximum(
                            hd_buf[i, slo] + hs_buf[i, slo] + e_lo, 0.0)
                        hd_buf[i, shi] = jnp.maximum(
                            hd_buf[i, shi] + hs_buf[i, shi] + e_hi, 0.0)
                    return 0
                lax.fori_loop(0, _CH, _row, 0)
                pltpu.async_copy(hd_buf, acc_sh.at[idx_d], sem_s, add=True)

            def _drain_scatter(st):
                idx_d, hd_buf, sem_s = st[0], st[2], st[7]
                pltpu.make_async_copy(hd_buf, acc_sh.at[idx_d], sem_s).wait()

            zsrc = stage[0][2]

            def _zero_body(i, _):
                for j in range(nvec):
                    zsrc[i, pl.ds(j * nl, nl)] = jnp.zeros((nl,), jnp.float32)
                return 0
            lax.fori_loop(0, zrows, _zero_body, 0)

            def _z(u, _):
                @pl.when(u % ns == s)
                def _():
                    pltpu.sync_copy(zsrc, acc_sh.at[pl.ds(u * zrows, zrows)])
                return 0
            lax.fori_loop(0, units, _z, 0)
            plsc.subcore_barrier()

            def _iter(g, r):
                n1 = (r + 1) % nst
                n2 = (r + 2) % nst

                @pl.when(g + 1 < chunks)
                def _():
                    _wait_idx(stage[n1])
                    _launch_gather(g + 1, stage[n1])
                _wait_gather(stage[r])
                _compute_scatter(stage[r])

                @pl.when((g >= 1) & (g + 2 < chunks))
                def _():
                    _drain_scatter(stage[n2])

                @pl.when(g + 2 < chunks)
                def _():
                    _launch_idx(g + 2, stage[n2])

            pltpu.sync_copy(dst.at[pl.ds(wid * ept, _CH)], stage[0][0])
            pltpu.sync_copy(src.at[pl.ds(wid * ept, _CH)], stage[0][1])
            _launch_gather(0, stage[0])
            _launch_idx(1, stage[1])

            def _triple(k, _):
                for off in range(nst):
                    _iter(nst * k + off, off)
                return 0
            lax.fori_loop(0, triples, _triple, 0)
            for off in range(rem):
                _iter(nst * triples + off, off)
            for g in range(chunks - 3, chunks):
                _drain_scatter(stage[g % nst])
            plsc.subcore_barrier()

            def _flush(u, _):
                @pl.when(u % ns == s)
                def _():
                    r0 = u * zrows
                    pltpu.sync_copy(acc_sh.at[pl.ds(r0, zrows)],
                                    out.at[t, c, pl.ds(r0, zrows)])
                return 0
            lax.fori_loop(0, units, _flush, 0)

        _process(0, hda, hsa, ea, dsta, srca)
        _process(1, hdb, hsb, eb, dstb, srcb)

    return sc_kernel(hd_a, hs_a, e_a, dst_a, src_a,
                     hd_b, hs_b, e_b, dst_b, src_b)


def kernel(x, xs_a, xs_b, edge_index_a, edge_index_b, edge_attr_a, edge_attr_b,
           W_dst_a, W_src_a, W_edge_a, b_a,
           W_dst_b, W_src_b, W_edge_b, b_b,
           W_u, b_u):
    d = x.shape[1]
    sig = jnp.asarray(_SIGMA)

    hd_a, hs_a, hd_b, hs_b = _node_transform(
        x, xs_a, xs_b, W_dst_a, W_src_a, W_dst_b, W_src_b)
    e_a, e_b = _edge_transform(
        edge_attr_a, edge_attr_b,
        W_edge_a[:, sig], W_edge_b[:, sig], b_a[sig], b_b[sig])
    e_a, e_b = _pack_bf16(e_a), _pack_bf16(e_b)

    dst_a = edge_index_a[0]
    src_a = edge_index_a[1]
    src_b = edge_index_b[0]
    dst_b = edge_index_b[1]

    acc = _sc_message_passing(hd_a, hs_a, e_a, dst_a, src_a,
                              hd_b, hs_b, e_b, dst_b, src_b)

    return _final_matmul(acc, W_u[:d], W_u[d:], b_u)


# speedup vs baseline: 2.8975x; 2.8975x over previous
import functools

import jax
import jax.numpy as jnp
from jax import lax
from jax.experimental import pallas as pl
from jax.experimental.pallas import tpu as pltpu
from jax.experimental.pallas import tpu_sc as plsc


def _node_body(x_ref, xsa_ref, xsb_ref, wda_ref, wsa_ref, wdb_ref, wsb_ref,
               hda_ref, hsa_ref, hdb_ref, hsb_ref):
    x = x_ref[...]
    hda_ref[...] = jnp.dot(x, wda_ref[...], preferred_element_type=jnp.float32)
    hdb_ref[...] = jnp.dot(x, wdb_ref[...], preferred_element_type=jnp.float32)
    hsa_ref[...] = jnp.dot(xsa_ref[...], wsa_ref[...],
                           preferred_element_type=jnp.float32)
    hsb_ref[...] = jnp.dot(xsb_ref[...], wsb_ref[...],
                           preferred_element_type=jnp.float32)


def _node_transform(x, xs_a, xs_b, Wda, Wsa, Wdb, Wsb):
    n, d = x.shape
    bn = 1000
    grid = (n // bn,)
    row_spec = pl.BlockSpec((bn, d), lambda i: (i, 0))
    w_spec = pl.BlockSpec((d, d), lambda i: (0, 0))
    out = jax.ShapeDtypeStruct((n, d), jnp.float32)
    return pl.pallas_call(
        _node_body,
        grid=grid,
        in_specs=[row_spec, row_spec, row_spec, w_spec, w_spec, w_spec, w_spec],
        out_specs=[row_spec, row_spec, row_spec, row_spec],
        out_shape=[out, out, out, out],
    )(x, xs_a, xs_b, Wda, Wsa, Wdb, Wsb)


def _edge_body(aa_ref, wea_ref, ba_ref, ea_ref):
    ea_ref[...] = (jnp.dot(aa_ref[...], wea_ref[...],
                           preferred_element_type=jnp.float32) + ba_ref[...])


def _edge_transform(attr, We, b):
    e, de = attr.shape
    d = We.shape[1]
    be = 8000
    grid = (e // be,)
    a_spec = pl.BlockSpec((be, de), lambda i: (i, 0))
    w_spec = pl.BlockSpec((de, d), lambda i: (0, 0))
    b_spec = pl.BlockSpec((1, d), lambda i: (0, 0))
    o_spec = pl.BlockSpec((be, d), lambda i: (i, 0))
    out = jax.ShapeDtypeStruct((e, d), jnp.float32)
    return pl.pallas_call(
        _edge_body,
        grid=grid,
        in_specs=[a_spec, w_spec, b_spec],
        out_specs=o_spec,
        out_shape=out,
    )(attr, We, b.reshape(1, d))


def _half_body(acc_ref, wu_ref, bu_ref, out_ref):
    msg = acc_ref[0] + acc_ref[1]
    out_ref[...] = (
        jnp.dot(msg, wu_ref[...], preferred_element_type=jnp.float32)
        + bu_ref[...])


def _half_matmul(acc, Wu, bu):
    n, d = acc.shape[1], acc.shape[2]
    bn = 1000
    grid = (n // bn,)
    return pl.pallas_call(
        _half_body,
        grid=grid,
        in_specs=[
            pl.BlockSpec((2, bn, d), lambda i: (0, i, 0)),
            pl.BlockSpec((d, d), lambda i: (0, 0)),
            pl.BlockSpec((1, d), lambda i: (0, 0)),
        ],
        out_specs=pl.BlockSpec((bn, d), lambda i: (i, 0)),
        out_shape=jax.ShapeDtypeStruct((n, d), jnp.float32),
    )(acc, Wu, bu.reshape(1, d))


def _half_add_body(acc_ref, wu_ref, prev_ref, out_ref):
    msg = acc_ref[0] + acc_ref[1]
    out_ref[...] = (
        jnp.dot(msg, wu_ref[...], preferred_element_type=jnp.float32)
        + prev_ref[...])


def _half_add_matmul(acc, Wu, prev):
    n, d = acc.shape[1], acc.shape[2]
    bn = 1000
    grid = (n // bn,)
    return pl.pallas_call(
        _half_add_body,
        grid=grid,
        in_specs=[
            pl.BlockSpec((2, bn, d), lambda i: (0, i, 0)),
            pl.BlockSpec((d, d), lambda i: (0, 0)),
            pl.BlockSpec((bn, d), lambda i: (i, 0)),
        ],
        out_specs=pl.BlockSpec((bn, d), lambda i: (i, 0)),
        out_shape=jax.ShapeDtypeStruct((n, d), jnp.float32),
    )(acc, Wu, prev)


_CH = 40


def _sc_message_passing(hd_in, hs_in, e_in, dst_in, src_in):
    n, d = hd_in.shape
    e = dst_in.shape[0]
    info = plsc.get_sparse_core_info()
    nc, ns, nl = info.num_cores, info.num_subcores, info.num_lanes
    nw = nc * ns
    ept = e // nw
    chunks = ept // _CH
    assert ept * nw == e and chunks * _CH == ept
    zrows = _CH
    units = n // zrows
    assert units * zrows == n
    nvec = d // nl

    assert chunks >= 4
    nst = 3
    triples = chunks // nst
    rem = chunks % nst

    mesh = plsc.VectorSubcoreMesh(core_axis_name="c", subcore_axis_name="s")

    buf_t = pltpu.VMEM((_CH, d), jnp.float32)
    idx_t = pltpu.VMEM((_CH,), jnp.int32)

    @functools.partial(
        pl.kernel,
        mesh=mesh,
        out_type=jax.ShapeDtypeStruct((2, n, d), jnp.float32),
        scratch_types=(
            [idx_t] * (2 * nst)
            + [buf_t] * (3 * nst)
            + [pltpu.VMEM_SHARED((n, d), jnp.float32)]
            + [pltpu.SemaphoreType.DMA] * (3 * nst)
        ),
    )
    def sc_kernel(hd, hs, ee, dst, src, out, *scr):
        idxs = scr[:2 * nst]
        bufs = scr[2 * nst:5 * nst]
        acc_sh = scr[5 * nst]
        sems = scr[5 * nst + 1:]
        stage = tuple(
            (idxs[2 * r], idxs[2 * r + 1],
             bufs[3 * r], bufs[3 * r + 1], bufs[3 * r + 2],
             sems[3 * r], sems[3 * r + 1], sems[3 * r + 2])
            for r in range(nst))
        c = lax.axis_index("c")
        s = lax.axis_index("s")
        wid = s * nc + c

        def _process():
            def _launch_idx(g, st):
                idx_d, idx_s = st[0], st[1]
                sem_i = st[6]
                base = wid * ept + g * _CH
                pltpu.async_copy(dst.at[pl.ds(base, _CH)], idx_d, sem_i)
                pltpu.async_copy(src.at[pl.ds(base, _CH)], idx_s, sem_i)

            def _wait_idx(st):
                idx_d, idx_s = st[0], st[1]
                sem_i = st[6]
                pltpu.make_async_copy(dst.at[pl.ds(0, _CH)], idx_d,
                                      sem_i).wait()
                pltpu.make_async_copy(src.at[pl.ds(0, _CH)], idx_s,
                                      sem_i).wait()

            def _launch_gather(g, st):
                idx_d, idx_s, hd_buf, hs_buf, e_buf = st[:5]
                sem_g = st[5]
                base = wid * ept + g * _CH
                pltpu.async_copy(hd.at[idx_d], hd_buf, sem_g)
                pltpu.async_copy(hs.at[idx_s], hs_buf, sem_g)
                pltpu.async_copy(ee.at[pl.ds(base, _CH)], e_buf, sem_g)

            def _wait_gather(st):
                idx_d, idx_s, hd_buf, hs_buf, e_buf = st[:5]
                sem_g = st[5]
                pltpu.make_async_copy(hd.at[idx_d], hd_buf, sem_g).wait()
                pltpu.make_async_copy(hs.at[idx_s], hs_buf, sem_g).wait()
                pltpu.make_async_copy(ee.at[pl.ds(0, _CH)], e_buf,
                                      sem_g).wait()

            def _compute_scatter(st):
                idx_d, _, hd_buf, hs_buf, e_buf = st[:5]
                sem_s = st[7]

                def _row(i, _):
                    for j in range(nvec):
                        sl = pl.ds(j * nl, nl)
                        m = hd_buf[i, sl] + hs_buf[i, sl] + e_buf[i, sl]
                        e_buf[i, sl] = jnp.maximum(m, 0.0)
                    return 0
                lax.fori_loop(0, _CH, _row, 0)
                pltpu.async_copy(e_buf, acc_sh.at[idx_d], sem_s, add=True)

            def _drain_scatter(st):
                idx_d, e_buf, sem_s = st[0], st[4], st[7]
                pltpu.make_async_copy(e_buf, acc_sh.at[idx_d], sem_s).wait()

            zsrc = stage[0][4]

            def _zero_body(i, _):
                for j in range(nvec):
                    zsrc[i, pl.ds(j * nl, nl)] = jnp.zeros((nl,), jnp.float32)
                return 0
            lax.fori_loop(0, zrows, _zero_body, 0)

            def _z(u, _):
                @pl.when(u % ns == s)
                def _():
                    pltpu.sync_copy(zsrc, acc_sh.at[pl.ds(u * zrows, zrows)])
                return 0
            lax.fori_loop(0, units, _z, 0)
            plsc.subcore_barrier()

            def _iter(g, r):
                n1 = (r + 1) % nst
                n2 = (r + 2) % nst

                @pl.when(g + 1 < chunks)
                def _():
                    _wait_idx(stage[n1])
                    _launch_gather(g + 1, stage[n1])
                _wait_gather(stage[r])
                _compute_scatter(stage[r])

                @pl.when((g >= 1) & (g + 2 < chunks))
                def _():
                    _drain_scatter(stage[n2])

                @pl.when(g + 2 < chunks)
                def _():
                    _launch_idx(g + 2, stage[n2])

            pltpu.sync_copy(dst.at[pl.ds(wid * ept, _CH)], stage[0][0])
            pltpu.sync_copy(src.at[pl.ds(wid * ept, _CH)], stage[0][1])
            _launch_gather(0, stage[0])
            _launch_idx(1, stage[1])

            def _triple(k, _):
                for off in range(nst):
                    _iter(nst * k + off, off)
                return 0
            lax.fori_loop(0, triples, _triple, 0)
            for off in range(rem):
                _iter(nst * triples + off, off)
            for g in range(chunks - 3, chunks):
                _drain_scatter(stage[g % nst])
            plsc.subcore_barrier()

            def _flush(u, _):
                @pl.when(u % ns == s)
                def _():
                    r0 = u * zrows
                    pltpu.sync_copy(acc_sh.at[pl.ds(r0, zrows)],
                                    out.at[c, pl.ds(r0, zrows)])
                return 0
            lax.fori_loop(0, units, _flush, 0)

        _process()

    return sc_kernel(hd_in, hs_in, e_in, dst_in, src_in)


def kernel(x, xs_a, xs_b, edge_index_a, edge_index_b, edge_attr_a, edge_attr_b,
           W_dst_a, W_src_a, W_edge_a, b_a,
           W_dst_b, W_src_b, W_edge_b, b_b,
           W_u, b_u):
    d = x.shape[1]

    hd_a, hs_a, hd_b, hs_b = _node_transform(
        x, xs_a, xs_b, W_dst_a, W_src_a, W_dst_b, W_src_b)

    dst_a = edge_index_a[0]
    src_a = edge_index_a[1]
    src_b = edge_index_b[0]
    dst_b = edge_index_b[1]

    e_a = _edge_transform(edge_attr_a, W_edge_a, b_a)
    acc_a = _sc_message_passing(hd_a, hs_a, e_a, dst_a, src_a)
    e_b = _edge_transform(edge_attr_b, W_edge_b, b_b)
    acc_b = _sc_message_passing(hd_b, hs_b, e_b, dst_b, src_b)
    out_a = _half_matmul(acc_a, W_u[:d], b_u)
    return _half_add_matmul(acc_b, W_u[d:], out_a)
